# baseline (speedup 1.0000x reference)
import functools

import jax
import jax.numpy as jnp
from jax.experimental import pallas as pl
from jax.experimental.pallas import tpu as pltpu

D_MODEL_ = 4096
N_EXPERTS_ = 64
TOP_K_ = 8
BT_ = 1024


def _router_block(x_ref, w_ref, vals_ref, idx_ref, loss_ref, acc_ref):
    i = pl.program_id(0)
    nsteps = pl.num_programs(0)

    @pl.when(i == 0)
    def _init():
        acc_ref[...] = jnp.zeros_like(acc_ref)

    logits = jax.lax.dot_general(
        w_ref[...], x_ref[...],
        dimension_numbers=(((1,), (1,)), ((), ())),
        preferred_element_type=jnp.float32,
    )

    m = jnp.max(logits, axis=0, keepdims=True)
    e = jnp.exp(logits - m)
    s = jnp.sum(e, axis=0, keepdims=True)
    probs = e / s

    iota_e = jax.lax.broadcasted_iota(jnp.int32, probs.shape, 0)
    work = probs
    vals_rows = []
    idx_rows = []
    for _ in range(TOP_K_):
        mx = jnp.max(work, axis=0, keepdims=True)
        cand = jnp.where(work == mx, iota_e, N_EXPERTS_)
        sel = jnp.min(cand, axis=0, keepdims=True)
        vals_rows.append(mx)
        idx_rows.append(sel)
        work = jnp.where(iota_e == sel, -1.0, work)

    vals8 = jnp.concatenate(vals_rows, axis=0)
    idx8 = jnp.concatenate(idx_rows, axis=0)
    vals_ref[...] = vals8.T
    idx_ref[...] = idx8.T

    sel_mask = (work < 0.0).astype(jnp.float32)
    cnt_part = jnp.sum(sel_mask, axis=1, keepdims=True)
    p_part = jnp.sum(probs, axis=1, keepdims=True)
    acc_ref[:, 0:1] += cnt_part
    acc_ref[:, 1:2] += p_part

    @pl.when(i == nsteps - 1)
    def _finish():
        n_tok = nsteps * BT_
        cnt = acc_ref[:, 0:1]
        ps = acc_ref[:, 1:2]
        scale = 1.0 / (float(n_tok) * float(TOP_K_) * float(n_tok))
        loss_ref[...] = (jnp.sum(cnt * ps) * scale).reshape(1, 1)


@functools.partial(jax.jit, static_argnames=())
def kernel(x, W):
    B, T, D = x.shape
    n_tok = B * T
    x2 = x.reshape(n_tok, D)
    grid = (n_tok // BT_,)
    vals, idx, loss = pl.pallas_call(
        _router_block,
        grid=grid,
        in_specs=[
            pl.BlockSpec((BT_, D), lambda i: (i, 0)),
            pl.BlockSpec((N_EXPERTS_, D), lambda i: (0, 0)),
        ],
        out_specs=[
            pl.BlockSpec((BT_, TOP_K_), lambda i: (i, 0)),
            pl.BlockSpec((BT_, TOP_K_), lambda i: (i, 0)),
            pl.BlockSpec((1, 1), lambda i: (0, 0)),
        ],
        out_shape=[
            jax.ShapeDtypeStruct((n_tok, TOP_K_), jnp.float32),
            jax.ShapeDtypeStruct((n_tok, TOP_K_), jnp.int32),
            jax.ShapeDtypeStruct((1, 1), jnp.float32),
        ],
        scratch_shapes=[pltpu.VMEM((N_EXPERTS_, 2), jnp.float32)],
    )(x2, W)
    return (vals.reshape(B, T, TOP_K_), idx.reshape(B, T, TOP_K_),
            loss.reshape(()))


# speedup vs baseline: 1.0031x; 1.0031x over previous
import functools

import jax
import jax.numpy as jnp
from jax.experimental import pallas as pl
from jax.experimental.pallas import tpu as pltpu

D_MODEL_ = 4096
N_EXPERTS_ = 64
TOP_K_ = 8
BT_ = 1024


def _router_block(x_ref, w_ref, vals_ref, idx_ref, loss_ref, acc_ref):
    i = pl.program_id(0) * pl.num_programs(1) + pl.program_id(1)
    nsteps = pl.num_programs(0) * pl.num_programs(1)

    @pl.when(i == 0)
    def _init():
        acc_ref[...] = jnp.zeros_like(acc_ref)

    logits = jax.lax.dot_general(
        w_ref[...], x_ref[0],
        dimension_numbers=(((1,), (1,)), ((), ())),
        preferred_element_type=jnp.float32,
    )

    m = jnp.max(logits, axis=0, keepdims=True)
    e = jnp.exp(logits - m)
    s = jnp.sum(e, axis=0, keepdims=True)
    probs = e / s

    iota_e = jax.lax.broadcasted_iota(jnp.int32, probs.shape, 0)
    work = probs
    vals_rows = []
    idx_rows = []
    for _ in range(TOP_K_):
        mx = jnp.max(work, axis=0, keepdims=True)
        cand = jnp.where(work == mx, iota_e, N_EXPERTS_)
        sel = jnp.min(cand, axis=0, keepdims=True)
        vals_rows.append(mx)
        idx_rows.append(sel)
        work = jnp.where(iota_e == sel, -1.0, work)

    vals8 = jnp.concatenate(vals_rows, axis=0)
    idx8 = jnp.concatenate(idx_rows, axis=0)
    vals_ref[...] = vals8.T[None]
    idx_ref[...] = idx8.T[None]

    sel_mask = (work < 0.0).astype(jnp.float32)
    cnt_part = jnp.sum(sel_mask, axis=1, keepdims=True)
    p_part = jnp.sum(probs, axis=1, keepdims=True)
    acc_ref[:, 0:1] += cnt_part
    acc_ref[:, 1:2] += p_part

    @pl.when(i == nsteps - 1)
    def _finish():
        n_tok = nsteps * BT_
        cnt = acc_ref[:, 0:1]
        ps = acc_ref[:, 1:2]
        scale = 1.0 / (float(n_tok) * float(TOP_K_) * float(n_tok))
        loss_ref[...] = (jnp.sum(cnt * ps) * scale).reshape(1, 1)


@functools.partial(jax.jit, static_argnames=())
def kernel(x, W):
    B, T, D = x.shape
    grid = (B, T // BT_)
    vals, idx, loss = pl.pallas_call(
        _router_block,
        grid=grid,
        in_specs=[
            pl.BlockSpec((1, BT_, D), lambda b, i: (b, i, 0)),
            pl.BlockSpec((N_EXPERTS_, D), lambda b, i: (0, 0)),
        ],
        out_specs=[
            pl.BlockSpec((1, BT_, TOP_K_), lambda b, i: (b, i, 0)),
            pl.BlockSpec((1, BT_, TOP_K_), lambda b, i: (b, i, 0)),
            pl.BlockSpec((1, 1), lambda b, i: (0, 0)),
        ],
        out_shape=[
            jax.ShapeDtypeStruct((B, T, TOP_K_), jnp.float32),
            jax.ShapeDtypeStruct((B, T, TOP_K_), jnp.int32),
            jax.ShapeDtypeStruct((1, 1), jnp.float32),
        ],
        scratch_shapes=[pltpu.VMEM((N_EXPERTS_, 2), jnp.float32)],
    )(x, W)
    return (vals, idx, loss.reshape(()))
